# Initial kernel scaffold; baseline (speedup 1.0000x reference)
import jax
import jax.numpy as jnp
from jax.experimental import pallas as pl


def kernel(seqs, context_cat_inputs, table):
    raise NotImplementedError("write your pallas kernel here")


# speedup vs baseline: 1.5813x; 1.5813x over previous
import functools

import jax
import jax.numpy as jnp
from jax import lax
from jax.experimental import pallas as pl
from jax.experimental.pallas import tpu as pltpu
from jax.experimental.pallas import tpu_sc as plsc

B, L, F = 1024, 50, 64
N_BAG = 26
ROWS = B * L

NC, NS = 2, 16
NW = NC * NS
ROWS_PER_W = ROWS // NW
GRP = 4
IDX_PER_GRP = GRP * N_BAG
GRPS_PER_W = ROWS_PER_W // GRP
IDX_ROWS = ROWS * N_BAG // IDX_PER_GRP


def _sc_body(idx_hbm, table_hbm, out_hbm, idx_v, rows_v, out_v, gsem):
    wid = lax.axis_index("s") * NC + lax.axis_index("c")
    pltpu.sync_copy(idx_hbm.at[pl.ds(wid * GRPS_PER_W, GRPS_PER_W)], idx_v)

    def unit(u, _):
        pltpu.async_copy(table_hbm.at[idx_v.at[u]], rows_v, gsem).wait()
        for r in range(GRP):
            for k in range(F // 16):
                acc = rows_v[r * N_BAG, pl.ds(k * 16, 16)]
                for j in range(1, N_BAG):
                    acc = acc + rows_v[r * N_BAG + j, pl.ds(k * 16, 16)]
                out_v[r, pl.ds(k * 16, 16)] = acc
        pltpu.sync_copy(out_v, out_hbm.at[pl.ds(wid * ROWS_PER_W + u * GRP, GRP)])
        return 0

    lax.fori_loop(0, GRPS_PER_W, unit, 0)


@functools.partial(jax.jit, static_argnums=())
def _embedding_bag(idx, table):
    mesh = plsc.VectorSubcoreMesh(core_axis_name="c", subcore_axis_name="s")
    k = functools.partial(
        pl.kernel,
        mesh=mesh,
        out_type=jax.ShapeDtypeStruct((ROWS, F), jnp.float32),
        scratch_types=[
            pltpu.VMEM((GRPS_PER_W, IDX_PER_GRP), jnp.int32),
            pltpu.VMEM((IDX_PER_GRP, F), jnp.float32),
            pltpu.VMEM((GRP, F), jnp.float32),
            pltpu.SemaphoreType.DMA,
        ],
        compiler_params=pltpu.CompilerParams(use_tc_tiling_on_sc=False),
    )(_sc_body)
    return k(idx, table)


def kernel(seqs, context_cat_inputs, table):
    b, l, f = seqs.shape
    idx = context_cat_inputs.astype(jnp.int32).reshape(IDX_ROWS, IDX_PER_GRP)
    out = _embedding_bag(idx, table)
    return out.reshape(b, l, f)


# speedup vs baseline: 1.9134x; 1.2100x over previous
import functools

import jax
import jax.numpy as jnp
from jax import lax
from jax.experimental import pallas as pl
from jax.experimental.pallas import tpu as pltpu
from jax.experimental.pallas import tpu_sc as plsc

B, L, F = 1024, 50, 64
N_BAG = 26
ROWS = B * L

NC, NS = 2, 16
NW = NC * NS
ROWS_PER_W = ROWS // NW
GRP = 4
IDX_PER_GRP = GRP * N_BAG
GRPS_PER_W = ROWS_PER_W // GRP
IDX_ROWS = ROWS * N_BAG // IDX_PER_GRP


NBUF = 4


def _sc_body(idx_hbm, table_hbm, out_hbm, idx_v, rows_v, out_v, gsem, osem):
    wid = lax.axis_index("s") * NC + lax.axis_index("c")
    base_row = wid * ROWS_PER_W
    pltpu.sync_copy(idx_hbm.at[pl.ds(wid * GRPS_PER_W, GRPS_PER_W)], idx_v)

    def gather(u, b):
        return pltpu.make_async_copy(
            table_hbm.at[idx_v.at[u]], rows_v.at[b], gsem.at[b])

    def out_write(u, b):
        return pltpu.make_async_copy(
            out_v.at[b], out_hbm.at[pl.ds(base_row + u * GRP, GRP)], osem.at[b])

    for b in range(NBUF):
        gather(b, b).start()

    def outer(t, _):
        u0 = t * NBUF
        for b in range(NBUF):
            u = u0 + b
            gather(u, b).wait()

            @pl.when(t > 0)
            def _():
                out_write(u - NBUF, b).wait()

            for r in range(GRP):
                for k in range(F // 16):
                    acc = rows_v[b, r * N_BAG, pl.ds(k * 16, 16)]
                    for j in range(1, N_BAG):
                        acc = acc + rows_v[b, r * N_BAG + j, pl.ds(k * 16, 16)]
                    out_v[b, r, pl.ds(k * 16, 16)] = acc

            @pl.when(u + NBUF < GRPS_PER_W)
            def _():
                gather(u + NBUF, b).start()

            out_write(u, b).start()
        return 0

    lax.fori_loop(0, GRPS_PER_W // NBUF, outer, 0)

    for b in range(NBUF):
        out_write(GRPS_PER_W - NBUF + b, b).wait()


@functools.partial(jax.jit, static_argnums=())
def _embedding_bag(idx, table):
    mesh = plsc.VectorSubcoreMesh(core_axis_name="c", subcore_axis_name="s")
    k = functools.partial(
        pl.kernel,
        mesh=mesh,
        out_type=jax.ShapeDtypeStruct((ROWS, F), jnp.float32),
        scratch_types=[
            pltpu.VMEM((GRPS_PER_W, IDX_PER_GRP), jnp.int32),
            pltpu.VMEM((NBUF, IDX_PER_GRP, F), jnp.float32),
            pltpu.VMEM((NBUF, GRP, F), jnp.float32),
            pltpu.SemaphoreType.DMA((NBUF,)),
            pltpu.SemaphoreType.DMA((NBUF,)),
        ],
        compiler_params=pltpu.CompilerParams(use_tc_tiling_on_sc=False),
    )(_sc_body)
    return k(idx, table)


def kernel(seqs, context_cat_inputs, table):
    b, l, f = seqs.shape
    idx = context_cat_inputs.astype(jnp.int32).reshape(IDX_ROWS, IDX_PER_GRP)
    out = _embedding_bag(idx, table)
    return out.reshape(b, l, f)
